# baseline (speedup 1.0000x reference)
import functools

import jax
import jax.numpy as jnp
from jax import lax
from jax.experimental import pallas as pl
from jax.experimental.pallas import tpu as pltpu
from jax.experimental.pallas import tpu_sc as plsc

N = 10000
E = 320000
F_IN = 128
H = 64

NC = 2
NS = 16
NW = NC * NS
CH = 128
EPW_RAW = E // NW
NFC = EPW_RAW // CH
REM = EPW_RAW - NFC * CH
NROWS = 10240
DW = 16


@functools.cache
def _mesh():
    return plsc.VectorSubcoreMesh(
        core_axis_name="c", subcore_axis_name="s",
        num_cores=NC, num_subcores=NS)


def _maybe_when(cond, fn):
    if isinstance(cond, bool):
        if cond:
            fn()
    else:
        pl.when(cond)(fn)


def _fill_rows(ref, nrows, width, value):
    def row(i, _):
        for k in range(width // 16):
            ref[i, pl.ds(k * 16, 16)] = jnp.full((16,), value, jnp.float32)
        return 0
    lax.fori_loop(0, nrows, row, 0)


def _zero_acc(acc_sh, zb_v, sid, width):
    base = sid * (NROWS // NS)
    def cp(t, _):
        pltpu.sync_copy(zb_v, acc_sh.at[pl.ds(base + t * 64, 64)])
        return 0
    lax.fori_loop(0, (NROWS // NS) // 64, cp, 0)


@functools.cache
def _deg_kernel():
    return pl.kernel(
        _deg_body,
        out_type=jax.ShapeDtypeStruct((NC * NROWS, DW), jnp.float32),
        mesh=_mesh(),
        scratch_types=[
            pltpu.VMEM((8, CH), jnp.int32),
            pltpu.VMEM((CH, DW), jnp.float32),
            pltpu.VMEM((64, DW), jnp.float32),
            pltpu.VMEM_SHARED((NROWS, DW), jnp.float32),
            [pltpu.SemaphoreType.DMA] * 4,
            [pltpu.SemaphoreType.DMA] * 8,
        ],
        compiler_params=pltpu.CompilerParams(use_tc_tiling_on_sc=False),
    )


def _deg_body(ei_hbm, out_hbm, idx_v, ones_v, zb_v, acc_sh, ssems, isems):
    cid = lax.axis_index("c")
    sid = lax.axis_index("s")
    wid = sid * NC + cid
    ebase = E + wid * EPW_RAW

    _fill_rows(ones_v, CH, DW, 1.0)
    _fill_rows(zb_v, 64, DW, 0.0)
    _zero_acc(acc_sh, zb_v, sid, DW)
    plsc.subcore_barrier()

    def start_idx(jj, s):
        pltpu.async_copy(ei_hbm.at[pl.ds(ebase + jj * CH, CH)],
                         idx_v.at[s], isems[s])

    def wait_idx(jj, s):
        pltpu.make_async_copy(ei_hbm.at[pl.ds(ebase + jj * CH, CH)],
                              idx_v.at[s], isems[s]).wait()

    def start_scatter(r, s):
        pltpu.async_copy(ones_v, acc_sh.at[idx_v.at[s]], ssems[r], add=True)

    def wait_scatter(r, s):
        pltpu.make_async_copy(
            ones_v, acc_sh.at[idx_v.at[s]], ssems[r]).wait()

    for s in range(6):
        start_idx(s, s)

    def step(jj, statics):
        r, s = statics
        wait_idx(jj, s)
        start_scatter(r, s)
        _maybe_when(jj >= 2,
                    lambda: wait_scatter((r + 2) % 4, (s + 6) % 8))
        _maybe_when(jj + 6 < NFC,
                    lambda: start_idx(jj + 6, (s + 6) % 8))

    def oct_(t, _):
        j = t * 8
        for b in range(8):
            step(j + b, (b % 4, b))
        return 0
    lax.fori_loop(0, 72 // 8, oct_, 0)
    for jj in range(72, NFC):
        step(jj, (jj % 4, jj % 8))
    wait_scatter((NFC - 2) % 4, (NFC - 2) % 8)
    wait_scatter((NFC - 1) % 4, (NFC - 1) % 8)

    pltpu.sync_copy(ei_hbm.at[pl.ds(ebase + NFC * CH, REM)],
                    idx_v.at[0, pl.ds(0, REM)])
    pltpu.sync_copy(ones_v.at[pl.ds(0, REM)],
                    acc_sh.at[idx_v.at[0, pl.ds(0, REM)]], add=True)

    plsc.subcore_barrier()
    base = sid * (NROWS // NS)
    pltpu.sync_copy(
        acc_sh.at[pl.ds(base, NROWS // NS)],
        out_hbm.at[pl.ds(cid * NROWS + base, NROWS // NS)],
    )


@functools.cache
def _agg_kernel():
    return pl.kernel(
        _agg_body,
        out_type=jax.ShapeDtypeStruct((NC * NROWS, H), jnp.float32),
        mesh=_mesh(),
        scratch_types=[
            pltpu.VMEM((16, CH), jnp.int32),
            pltpu.VMEM((4, CH, H), jnp.float32),
            pltpu.VMEM((REM, H), jnp.float32),
            pltpu.VMEM((64, H), jnp.float32),
            pltpu.VMEM_SHARED((NROWS, H), jnp.float32),
            pltpu.VMEM_SHARED((N, H), jnp.float32),
            [pltpu.SemaphoreType.DMA] * 4,
            [pltpu.SemaphoreType.DMA] * 4,
            [pltpu.SemaphoreType.DMA] * 8,
        ],
        compiler_params=pltpu.CompilerParams(use_tc_tiling_on_sc=False),
    )


def _agg_body(table_hbm, ei_hbm, out_hbm,
              idx_v, rows_v, rem_v, zb_v, acc_sh, tbl_sh,
              gsems, ssems, isems):
    cid = lax.axis_index("c")
    sid = lax.axis_index("s")
    wid = sid * NC + cid
    ebase = wid * EPW_RAW

    _fill_rows(zb_v, 64, H, 0.0)
    _zero_acc(acc_sh, zb_v, sid, H)
    tbase = sid * (N // NS)
    pltpu.sync_copy(table_hbm.at[pl.ds(tbase, N // NS)],
                    tbl_sh.at[pl.ds(tbase, N // NS)])
    plsc.subcore_barrier()

    def start_idx(jj, s):
        pltpu.async_copy(ei_hbm.at[pl.ds(ebase + jj * CH, CH)],
                         idx_v.at[2 * s], isems[s])
        pltpu.async_copy(ei_hbm.at[pl.ds(E + ebase + jj * CH, CH)],
                         idx_v.at[2 * s + 1], isems[s])

    def wait_idx(jj, s):
        pltpu.make_async_copy(ei_hbm.at[pl.ds(ebase + jj * CH, CH)],
                              idx_v.at[2 * s], isems[s]).wait()
        pltpu.make_async_copy(ei_hbm.at[pl.ds(E + ebase + jj * CH, CH)],
                              idx_v.at[2 * s + 1], isems[s]).wait()

    def start_gather(r, s):
        pltpu.async_copy(tbl_sh.at[idx_v.at[2 * s]], rows_v.at[r], gsems[r])

    def wait_gather(r, s):
        pltpu.make_async_copy(
            tbl_sh.at[idx_v.at[2 * s]], rows_v.at[r], gsems[r]).wait()

    def start_scatter(r, s):
        pltpu.async_copy(
            rows_v.at[r], acc_sh.at[idx_v.at[2 * s + 1]], ssems[r], add=True)

    def wait_scatter(r, s):
        pltpu.make_async_copy(
            rows_v.at[r], acc_sh.at[idx_v.at[2 * s + 1]], ssems[r]).wait()

    for s in range(6):
        start_idx(s, s)
    for b in range(2):
        wait_idx(b, b)
        start_gather(b, b)

    def step(jj, statics):
        r, s = statics
        rn, sn, sf = (r + 2) % 4, (s + 2) % 8, (s + 6) % 8
        wait_gather(r, s)
        start_scatter(r, s)
        _maybe_when(jj >= 2, lambda: wait_scatter(rn, sf))
        _maybe_when(jj + 6 < NFC, lambda: start_idx(jj + 6, sf))

        def _pref():
            wait_idx(jj + 2, sn)
            start_gather(rn, sn)
        _maybe_when(jj + 2 < NFC, _pref)

    def oct_(t, _):
        j = t * 8
        for b in range(8):
            step(j + b, (b % 4, b))
        return 0
    lax.fori_loop(0, 72 // 8, oct_, 0)
    for jj in range(72, NFC):
        step(jj, (jj % 4, jj % 8))

    wait_scatter((NFC - 2) % 4, (NFC - 2) % 8)
    wait_scatter((NFC - 1) % 4, (NFC - 1) % 8)

    pltpu.sync_copy(ei_hbm.at[pl.ds(ebase + NFC * CH, REM)],
                    idx_v.at[0, pl.ds(0, REM)])
    pltpu.sync_copy(ei_hbm.at[pl.ds(E + ebase + NFC * CH, REM)],
                    idx_v.at[1, pl.ds(0, REM)])
    pltpu.sync_copy(tbl_sh.at[idx_v.at[0, pl.ds(0, REM)]], rem_v)
    pltpu.sync_copy(rem_v, acc_sh.at[idx_v.at[1, pl.ds(0, REM)]], add=True)

    plsc.subcore_barrier()
    base = sid * (NROWS // NS)
    pltpu.sync_copy(
        acc_sh.at[pl.ds(base, NROWS // NS)],
        out_hbm.at[pl.ds(cid * NROWS + base, NROWS // NS)],
    )


def _tc_call(body, out_shapes, *args):
    return pl.pallas_call(
        body,
        out_shape=out_shapes,
    )(*args)


def _tc0_body(x_ref, w1_ref, hw_ref):
    hw_ref[...] = jnp.dot(x_ref[...], w1_ref[...],
                          preferred_element_type=jnp.float32)


def _tc1_body(hw_ref, degp_ref, dinv_ref, hws_ref):
    deg = (degp_ref[0:N, 0:1]
           + degp_ref[NROWS:NROWS + N, 0:1] + 1.0)
    dinv = lax.rsqrt(deg)
    dinv_ref[...] = dinv
    hws_ref[...] = hw_ref[...] * dinv


def _tc_mid_body(parts_ref, hws_ref, dinv_ref, b_ref, w_ref, out_ref):
    dinv = dinv_ref[...]
    agg = parts_ref[0:N, :] + parts_ref[NROWS:NROWS + N, :] + hws_ref[...]
    h = jnp.maximum(agg * dinv + b_ref[...], 0.0)
    out_ref[...] = jnp.dot(h, w_ref[...],
                           preferred_element_type=jnp.float32) * dinv


def _sigmoid(x):
    return 1.0 / (1.0 + jnp.exp(-x))


def _tc_fin_body(parts_ref, hws_ref, dinv_ref, b3_ref,
                 wc1_ref, bc1_ref, wc2_ref, bc2_ref,
                 wr1_ref, br1_ref, wr2_ref, br2_ref,
                 h_ref, causal_ref, risk_ref):
    dinv = dinv_ref[...]
    agg = parts_ref[0:N, :] + parts_ref[NROWS:NROWS + N, :] + hws_ref[...]
    h = agg * dinv + b3_ref[...]
    h_ref[...] = h
    c1 = jnp.maximum(jnp.dot(h, wc1_ref[...],
                             preferred_element_type=jnp.float32)
                     + bc1_ref[...], 0.0)
    causal_ref[...] = _sigmoid(
        jnp.sum(c1 * wc2_ref[...], axis=1, keepdims=True) + bc2_ref[...])
    r1 = jnp.maximum(jnp.dot(h, wr1_ref[...],
                             preferred_element_type=jnp.float32)
                     + br1_ref[...], 0.0)
    risk_ref[...] = _sigmoid(
        jnp.sum(r1 * wr2_ref[...], axis=1, keepdims=True) + br2_ref[...])


def kernel(x, edge_index, W1, b1, W2, b2, W3, b3,
           Wc1, bc1, Wc2, bc2, Wr1, br1, Wr2, br2):
    ei = edge_index.reshape(2 * E)

    b1r = b1.reshape(1, H)
    b2r = b2.reshape(1, H)
    b3r = b3.reshape(1, H)
    bc1r = bc1.reshape(1, H // 2)
    bc2r = bc2.reshape(1, 1)
    br1r = br1.reshape(1, H // 2)
    br2r = br2.reshape(1, 1)
    wc2r = Wc2.reshape(1, H // 2)
    wr2r = Wr2.reshape(1, H // 2)

    degp = _deg_kernel()(ei)

    hw1 = _tc_call(
        _tc0_body,
        jax.ShapeDtypeStruct((N, H), jnp.float32),
        x, W1)
    dinv, hws1 = _tc_call(
        _tc1_body,
        (jax.ShapeDtypeStruct((N, 1), jnp.float32),
         jax.ShapeDtypeStruct((N, H), jnp.float32)),
        hw1, degp)

    parts1 = _agg_kernel()(hws1, ei)
    hws2 = _tc_call(
        _tc_mid_body,
        jax.ShapeDtypeStruct((N, H), jnp.float32),
        parts1, hws1, dinv, b1r, W2)

    parts2 = _agg_kernel()(hws2, ei)
    hws3 = _tc_call(
        _tc_mid_body,
        jax.ShapeDtypeStruct((N, H), jnp.float32),
        parts2, hws2, dinv, b2r, W3)

    parts3 = _agg_kernel()(hws3, ei)
    h, causal, risk = _tc_call(
        _tc_fin_body,
        (jax.ShapeDtypeStruct((N, H), jnp.float32),
         jax.ShapeDtypeStruct((N, 1), jnp.float32),
         jax.ShapeDtypeStruct((N, 1), jnp.float32)),
        parts3, hws3, dinv, b3r,
        Wc1, bc1r, wc2r, bc2r, Wr1, br1r, wr2r, br2r)

    return (h, causal, risk)


# speedup vs baseline: 1.0216x; 1.0216x over previous
import functools

import jax
import jax.numpy as jnp
from jax import lax
from jax.experimental import pallas as pl
from jax.experimental.pallas import tpu as pltpu
from jax.experimental.pallas import tpu_sc as plsc

N = 10000
E = 320000
F_IN = 128
H = 64

NC = 2
NS = 16
NW = NC * NS
CH = 128
EPW_RAW = E // NW
NFC = EPW_RAW // CH
REM = EPW_RAW - NFC * CH
NROWS = 10240
DW = 16


@functools.cache
def _mesh():
    return plsc.VectorSubcoreMesh(
        core_axis_name="c", subcore_axis_name="s",
        num_cores=NC, num_subcores=NS)


def _maybe_when(cond, fn):
    if isinstance(cond, bool):
        if cond:
            fn()
    else:
        pl.when(cond)(fn)


def _fill_rows(ref, nrows, width, value):
    def row(i, _):
        for k in range(width // 16):
            ref[i, pl.ds(k * 16, 16)] = jnp.full((16,), value, jnp.float32)
        return 0
    lax.fori_loop(0, nrows, row, 0)


def _zero_acc(acc_sh, zb_v, sid, width):
    base = sid * (NROWS // NS)
    def cp(t, _):
        pltpu.sync_copy(zb_v, acc_sh.at[pl.ds(base + t * 64, 64)])
        return 0
    lax.fori_loop(0, (NROWS // NS) // 64, cp, 0)


@functools.cache
def _deg_kernel():
    return pl.kernel(
        _deg_body,
        out_type=jax.ShapeDtypeStruct((NC * NROWS, DW), jnp.float32),
        mesh=_mesh(),
        scratch_types=[
            pltpu.VMEM((8, CH), jnp.int32),
            pltpu.VMEM((CH, DW), jnp.float32),
            pltpu.VMEM((64, DW), jnp.float32),
            pltpu.VMEM_SHARED((NROWS, DW), jnp.float32),
            [pltpu.SemaphoreType.DMA] * 4,
            [pltpu.SemaphoreType.DMA] * 8,
        ],
        compiler_params=pltpu.CompilerParams(use_tc_tiling_on_sc=False),
    )


def _deg_body(ei_hbm, out_hbm, idx_v, ones_v, zb_v, acc_sh, ssems, isems):
    cid = lax.axis_index("c")
    sid = lax.axis_index("s")
    wid = sid * NC + cid
    ebase = E + wid * EPW_RAW

    _fill_rows(ones_v, CH, DW, 1.0)
    _fill_rows(zb_v, 64, DW, 0.0)
    _zero_acc(acc_sh, zb_v, sid, DW)
    plsc.subcore_barrier()

    def start_idx(jj, s):
        pltpu.async_copy(ei_hbm.at[pl.ds(ebase + jj * CH, CH)],
                         idx_v.at[s], isems[s])

    def wait_idx(jj, s):
        pltpu.make_async_copy(ei_hbm.at[pl.ds(ebase + jj * CH, CH)],
                              idx_v.at[s], isems[s]).wait()

    def start_scatter(r, s):
        pltpu.async_copy(ones_v, acc_sh.at[idx_v.at[s]], ssems[r], add=True)

    def wait_scatter(r, s):
        pltpu.make_async_copy(
            ones_v, acc_sh.at[idx_v.at[s]], ssems[r]).wait()

    for s in range(6):
        start_idx(s, s)

    def step(jj, statics):
        r, s = statics
        wait_idx(jj, s)
        start_scatter(r, s)
        _maybe_when(jj >= 2,
                    lambda: wait_scatter((r + 2) % 4, (s + 6) % 8))
        _maybe_when(jj + 6 < NFC,
                    lambda: start_idx(jj + 6, (s + 6) % 8))

    def oct_(t, _):
        j = t * 8
        for b in range(8):
            step(j + b, (b % 4, b))
        return 0
    lax.fori_loop(0, 72 // 8, oct_, 0)
    for jj in range(72, NFC):
        step(jj, (jj % 4, jj % 8))
    wait_scatter((NFC - 2) % 4, (NFC - 2) % 8)
    wait_scatter((NFC - 1) % 4, (NFC - 1) % 8)

    pltpu.sync_copy(ei_hbm.at[pl.ds(ebase + NFC * CH, REM)],
                    idx_v.at[0, pl.ds(0, REM)])
    pltpu.sync_copy(ones_v.at[pl.ds(0, REM)],
                    acc_sh.at[idx_v.at[0, pl.ds(0, REM)]], add=True)

    plsc.subcore_barrier()
    base = sid * (NROWS // NS)
    pltpu.sync_copy(
        acc_sh.at[pl.ds(base, NROWS // NS)],
        out_hbm.at[pl.ds(cid * NROWS + base, NROWS // NS)],
    )


@functools.cache
def _agg_kernel():
    return pl.kernel(
        _agg_body,
        out_type=jax.ShapeDtypeStruct((NC * NROWS, H), jnp.float32),
        mesh=_mesh(),
        scratch_types=[
            pltpu.VMEM((16, CH), jnp.int32),
            pltpu.VMEM((4, CH, H), jnp.float32),
            pltpu.VMEM((REM, H), jnp.float32),
            pltpu.VMEM((64, H), jnp.float32),
            pltpu.VMEM_SHARED((NROWS, H), jnp.float32),
            pltpu.VMEM_SHARED((N, H), jnp.float32),
            [pltpu.SemaphoreType.DMA] * 4,
            [pltpu.SemaphoreType.DMA] * 4,
            [pltpu.SemaphoreType.DMA] * 8,
        ],
        compiler_params=pltpu.CompilerParams(use_tc_tiling_on_sc=False),
    )


def _agg_body(table_hbm, ei_hbm, out_hbm,
              idx_v, rows_v, rem_v, zb_v, acc_sh, tbl_sh,
              gsems, ssems, isems):
    cid = lax.axis_index("c")
    sid = lax.axis_index("s")
    wid = sid * NC + cid
    ebase = wid * EPW_RAW

    _fill_rows(zb_v, 64, H, 0.0)
    tbase = sid * (N // NS)
    pltpu.async_copy(table_hbm.at[pl.ds(tbase, N // NS)],
                     tbl_sh.at[pl.ds(tbase, N // NS)], gsems[1])
    zbase = sid * (NROWS // NS)
    for t in range((NROWS // NS) // 64):
        pltpu.async_copy(zb_v, acc_sh.at[pl.ds(zbase + t * 64, 64)],
                         gsems[0])
    for t in range((NROWS // NS) // 64):
        pltpu.make_async_copy(zb_v, acc_sh.at[pl.ds(zbase + t * 64, 64)],
                              gsems[0]).wait()
    pltpu.make_async_copy(table_hbm.at[pl.ds(tbase, N // NS)],
                          tbl_sh.at[pl.ds(tbase, N // NS)], gsems[1]).wait()
    plsc.subcore_barrier()

    def start_idx(jj, s):
        pltpu.async_copy(ei_hbm.at[pl.ds(ebase + jj * CH, CH)],
                         idx_v.at[2 * s], isems[s])
        pltpu.async_copy(ei_hbm.at[pl.ds(E + ebase + jj * CH, CH)],
                         idx_v.at[2 * s + 1], isems[s])

    def wait_idx(jj, s):
        pltpu.make_async_copy(ei_hbm.at[pl.ds(ebase + jj * CH, CH)],
                              idx_v.at[2 * s], isems[s]).wait()
        pltpu.make_async_copy(ei_hbm.at[pl.ds(E + ebase + jj * CH, CH)],
                              idx_v.at[2 * s + 1], isems[s]).wait()

    def start_gather(r, s):
        pltpu.async_copy(tbl_sh.at[idx_v.at[2 * s]], rows_v.at[r], gsems[r])

    def wait_gather(r, s):
        pltpu.make_async_copy(
            tbl_sh.at[idx_v.at[2 * s]], rows_v.at[r], gsems[r]).wait()

    def start_scatter(r, s):
        pltpu.async_copy(
            rows_v.at[r], acc_sh.at[idx_v.at[2 * s + 1]], ssems[r], add=True)

    def wait_scatter(r, s):
        pltpu.make_async_copy(
            rows_v.at[r], acc_sh.at[idx_v.at[2 * s + 1]], ssems[r]).wait()

    for s in range(6):
        start_idx(s, s)
    for b in range(2):
        wait_idx(b, b)
        start_gather(b, b)

    def step(jj, statics):
        r, s = statics
        rn, sn, sf = (r + 2) % 4, (s + 2) % 8, (s + 6) % 8
        wait_gather(r, s)
        start_scatter(r, s)
        _maybe_when(jj >= 2, lambda: wait_scatter(rn, sf))
        _maybe_when(jj + 6 < NFC, lambda: start_idx(jj + 6, sf))

        def _pref():
            wait_idx(jj + 2, sn)
            start_gather(rn, sn)
        _maybe_when(jj + 2 < NFC, _pref)

    def oct_(t, _):
        j = t * 8
        for b in range(8):
            step(j + b, (b % 4, b))
        return 0
    lax.fori_loop(0, 72 // 8, oct_, 0)
    for jj in range(72, NFC):
        step(jj, (jj % 4, jj % 8))

    wait_scatter((NFC - 2) % 4, (NFC - 2) % 8)
    wait_scatter((NFC - 1) % 4, (NFC - 1) % 8)

    pltpu.sync_copy(ei_hbm.at[pl.ds(ebase + NFC * CH, REM)],
                    idx_v.at[0, pl.ds(0, REM)])
    pltpu.sync_copy(ei_hbm.at[pl.ds(E + ebase + NFC * CH, REM)],
                    idx_v.at[1, pl.ds(0, REM)])
    pltpu.sync_copy(tbl_sh.at[idx_v.at[0, pl.ds(0, REM)]], rem_v)
    pltpu.sync_copy(rem_v, acc_sh.at[idx_v.at[1, pl.ds(0, REM)]], add=True)

    plsc.subcore_barrier()
    base = sid * (NROWS // NS)
    pltpu.sync_copy(
        acc_sh.at[pl.ds(base, NROWS // NS)],
        out_hbm.at[pl.ds(cid * NROWS + base, NROWS // NS)],
    )


def _tc_call(body, out_shapes, *args):
    return pl.pallas_call(
        body,
        out_shape=out_shapes,
    )(*args)


def _tc1_body(x_ref, w1_ref, degp_ref, dinv_ref, hws_ref):
    deg = (degp_ref[0:N, 0:1]
           + degp_ref[NROWS:NROWS + N, 0:1] + 1.0)
    dinv = lax.rsqrt(deg)
    dinv_ref[...] = dinv
    hw = jnp.dot(x_ref[...], w1_ref[...], preferred_element_type=jnp.float32)
    hws_ref[...] = hw * dinv


def _tc_mid_body(parts_ref, hws_ref, dinv_ref, b_ref, w_ref, out_ref):
    dinv = dinv_ref[...]
    agg = parts_ref[0:N, :] + parts_ref[NROWS:NROWS + N, :] + hws_ref[...]
    h = jnp.maximum(agg * dinv + b_ref[...], 0.0)
    out_ref[...] = jnp.dot(h, w_ref[...],
                           preferred_element_type=jnp.float32) * dinv


def _sigmoid(x):
    return 1.0 / (1.0 + jnp.exp(-x))


def _tc_fin_body(parts_ref, hws_ref, dinv_ref, b3_ref,
                 wc1_ref, bc1_ref, wc2_ref, bc2_ref,
                 wr1_ref, br1_ref, wr2_ref, br2_ref,
                 h_ref, causal_ref, risk_ref):
    dinv = dinv_ref[...]
    agg = parts_ref[0:N, :] + parts_ref[NROWS:NROWS + N, :] + hws_ref[...]
    h = agg * dinv + b3_ref[...]
    h_ref[...] = h
    c1 = jnp.maximum(jnp.dot(h, wc1_ref[...],
                             preferred_element_type=jnp.float32)
                     + bc1_ref[...], 0.0)
    causal_ref[...] = _sigmoid(
        jnp.sum(c1 * wc2_ref[...], axis=1, keepdims=True) + bc2_ref[...])
    r1 = jnp.maximum(jnp.dot(h, wr1_ref[...],
                             preferred_element_type=jnp.float32)
                     + br1_ref[...], 0.0)
    risk_ref[...] = _sigmoid(
        jnp.sum(r1 * wr2_ref[...], axis=1, keepdims=True) + br2_ref[...])


def kernel(x, edge_index, W1, b1, W2, b2, W3, b3,
           Wc1, bc1, Wc2, bc2, Wr1, br1, Wr2, br2):
    ei = edge_index.reshape(2 * E)

    b1r = b1.reshape(1, H)
    b2r = b2.reshape(1, H)
    b3r = b3.reshape(1, H)
    bc1r = bc1.reshape(1, H // 2)
    bc2r = bc2.reshape(1, 1)
    br1r = br1.reshape(1, H // 2)
    br2r = br2.reshape(1, 1)
    wc2r = Wc2.reshape(1, H // 2)
    wr2r = Wr2.reshape(1, H // 2)

    degp = _deg_kernel()(ei)

    dinv, hws1 = _tc_call(
        _tc1_body,
        (jax.ShapeDtypeStruct((N, 1), jnp.float32),
         jax.ShapeDtypeStruct((N, H), jnp.float32)),
        x, W1, degp)

    parts1 = _agg_kernel()(hws1, ei)
    hws2 = _tc_call(
        _tc_mid_body,
        jax.ShapeDtypeStruct((N, H), jnp.float32),
        parts1, hws1, dinv, b1r, W2)

    parts2 = _agg_kernel()(hws2, ei)
    hws3 = _tc_call(
        _tc_mid_body,
        jax.ShapeDtypeStruct((N, H), jnp.float32),
        parts2, hws2, dinv, b2r, W3)

    parts3 = _agg_kernel()(hws3, ei)
    h, causal, risk = _tc_call(
        _tc_fin_body,
        (jax.ShapeDtypeStruct((N, H), jnp.float32),
         jax.ShapeDtypeStruct((N, 1), jnp.float32),
         jax.ShapeDtypeStruct((N, 1), jnp.float32)),
        parts3, hws3, dinv, b3r,
        Wc1, bc1r, wc2r, bc2r, Wr1, br1r, wr2r, br2r)

    return (h, causal, risk)


# speedup vs baseline: 1.0251x; 1.0035x over previous
import functools

import jax
import jax.numpy as jnp
from jax import lax
from jax.experimental import pallas as pl
from jax.experimental.pallas import tpu as pltpu
from jax.experimental.pallas import tpu_sc as plsc

N = 10000
E = 320000
F_IN = 128
H = 64

NC = 2
NS = 16
NW = NC * NS
CH = 128
EPW_RAW = E // NW
NFC = EPW_RAW // CH
REM = EPW_RAW - NFC * CH
NROWS = 10240
DW = 16


@functools.cache
def _mesh():
    return plsc.VectorSubcoreMesh(
        core_axis_name="c", subcore_axis_name="s",
        num_cores=NC, num_subcores=NS)


def _maybe_when(cond, fn):
    if isinstance(cond, bool):
        if cond:
            fn()
    else:
        pl.when(cond)(fn)


def _fill_rows(ref, nrows, width, value):
    def row(i, _):
        for k in range(width // 16):
            ref[i, pl.ds(k * 16, 16)] = jnp.full((16,), value, jnp.float32)
        return 0
    lax.fori_loop(0, nrows, row, 0)


def _zero_acc(acc_sh, zb_v, sid, width):
    base = sid * (NROWS // NS)
    def cp(t, _):
        pltpu.sync_copy(zb_v, acc_sh.at[pl.ds(base + t * 64, 64)])
        return 0
    lax.fori_loop(0, (NROWS // NS) // 64, cp, 0)


@functools.cache
def _deg_kernel():
    return pl.kernel(
        _deg_body,
        out_type=jax.ShapeDtypeStruct((NC * NROWS, DW), jnp.float32),
        mesh=_mesh(),
        scratch_types=[
            pltpu.VMEM((8, CH), jnp.int32),
            pltpu.VMEM((CH, DW), jnp.float32),
            pltpu.VMEM((64, DW), jnp.float32),
            pltpu.VMEM_SHARED((NROWS, DW), jnp.float32),
            [pltpu.SemaphoreType.DMA] * 4,
            [pltpu.SemaphoreType.DMA] * 8,
        ],
        compiler_params=pltpu.CompilerParams(use_tc_tiling_on_sc=False),
    )


def _deg_body(ei_hbm, out_hbm, idx_v, ones_v, zb_v, acc_sh, ssems, isems):
    cid = lax.axis_index("c")
    sid = lax.axis_index("s")
    wid = sid * NC + cid
    ebase = E + wid * EPW_RAW

    _fill_rows(zb_v, 64, DW, 0.0)
    zbase = sid * (NROWS // NS)
    for t in range((NROWS // NS) // 64):
        pltpu.async_copy(zb_v, acc_sh.at[pl.ds(zbase + t * 64, 64)],
                         ssems[0])
    _fill_rows(ones_v, CH, DW, 1.0)
    for t in range((NROWS // NS) // 64):
        pltpu.make_async_copy(zb_v, acc_sh.at[pl.ds(zbase + t * 64, 64)],
                              ssems[0]).wait()
    plsc.subcore_barrier()

    def start_idx(jj, s):
        pltpu.async_copy(ei_hbm.at[pl.ds(ebase + jj * CH, CH)],
                         idx_v.at[s], isems[s])

    def wait_idx(jj, s):
        pltpu.make_async_copy(ei_hbm.at[pl.ds(ebase + jj * CH, CH)],
                              idx_v.at[s], isems[s]).wait()

    def start_scatter(r, s):
        pltpu.async_copy(ones_v, acc_sh.at[idx_v.at[s]], ssems[r], add=True)

    def wait_scatter(r, s):
        pltpu.make_async_copy(
            ones_v, acc_sh.at[idx_v.at[s]], ssems[r]).wait()

    for s in range(6):
        start_idx(s, s)

    def step(jj, statics):
        r, s = statics
        wait_idx(jj, s)
        start_scatter(r, s)
        _maybe_when(jj >= 2,
                    lambda: wait_scatter((r + 2) % 4, (s + 6) % 8))
        _maybe_when(jj + 6 < NFC,
                    lambda: start_idx(jj + 6, (s + 6) % 8))

    def oct_(t, _):
        j = t * 8
        for b in range(8):
            step(j + b, (b % 4, b))
        return 0
    lax.fori_loop(0, 72 // 8, oct_, 0)
    for jj in range(72, NFC):
        step(jj, (jj % 4, jj % 8))
    wait_scatter((NFC - 2) % 4, (NFC - 2) % 8)
    wait_scatter((NFC - 1) % 4, (NFC - 1) % 8)

    pltpu.sync_copy(ei_hbm.at[pl.ds(ebase + NFC * CH, REM)],
                    idx_v.at[0, pl.ds(0, REM)])
    pltpu.sync_copy(ones_v.at[pl.ds(0, REM)],
                    acc_sh.at[idx_v.at[0, pl.ds(0, REM)]], add=True)

    plsc.subcore_barrier()
    base = sid * (NROWS // NS)
    pltpu.sync_copy(
        acc_sh.at[pl.ds(base, NROWS // NS)],
        out_hbm.at[pl.ds(cid * NROWS + base, NROWS // NS)],
    )


@functools.cache
def _agg_kernel():
    return pl.kernel(
        _agg_body,
        out_type=jax.ShapeDtypeStruct((NC * NROWS, H), jnp.float32),
        mesh=_mesh(),
        scratch_types=[
            pltpu.VMEM((16, CH), jnp.int32),
            pltpu.VMEM((4, CH, H), jnp.float32),
            pltpu.VMEM((REM, H), jnp.float32),
            pltpu.VMEM((64, H), jnp.float32),
            pltpu.VMEM_SHARED((NROWS, H), jnp.float32),
            pltpu.VMEM_SHARED((N, H), jnp.float32),
            [pltpu.SemaphoreType.DMA] * 4,
            [pltpu.SemaphoreType.DMA] * 4,
            [pltpu.SemaphoreType.DMA] * 8,
        ],
        compiler_params=pltpu.CompilerParams(use_tc_tiling_on_sc=False),
    )


def _agg_body(table_hbm, ei_hbm, out_hbm,
              idx_v, rows_v, rem_v, zb_v, acc_sh, tbl_sh,
              gsems, ssems, isems):
    cid = lax.axis_index("c")
    sid = lax.axis_index("s")
    wid = sid * NC + cid
    ebase = wid * EPW_RAW

    _fill_rows(zb_v, 64, H, 0.0)
    tbase = sid * (N // NS)
    pltpu.async_copy(table_hbm.at[pl.ds(tbase, N // NS)],
                     tbl_sh.at[pl.ds(tbase, N // NS)], gsems[1])
    zbase = sid * (NROWS // NS)
    for t in range((NROWS // NS) // 64):
        pltpu.async_copy(zb_v, acc_sh.at[pl.ds(zbase + t * 64, 64)],
                         gsems[0])
    for t in range((NROWS // NS) // 64):
        pltpu.make_async_copy(zb_v, acc_sh.at[pl.ds(zbase + t * 64, 64)],
                              gsems[0]).wait()
    pltpu.make_async_copy(table_hbm.at[pl.ds(tbase, N // NS)],
                          tbl_sh.at[pl.ds(tbase, N // NS)], gsems[1]).wait()
    plsc.subcore_barrier()

    def start_idx(jj, s):
        pltpu.async_copy(ei_hbm.at[pl.ds(ebase + jj * CH, CH)],
                         idx_v.at[2 * s], isems[s])
        pltpu.async_copy(ei_hbm.at[pl.ds(E + ebase + jj * CH, CH)],
                         idx_v.at[2 * s + 1], isems[s])

    def wait_idx(jj, s):
        pltpu.make_async_copy(ei_hbm.at[pl.ds(ebase + jj * CH, CH)],
                              idx_v.at[2 * s], isems[s]).wait()
        pltpu.make_async_copy(ei_hbm.at[pl.ds(E + ebase + jj * CH, CH)],
                              idx_v.at[2 * s + 1], isems[s]).wait()

    def start_gather(r, s):
        pltpu.async_copy(tbl_sh.at[idx_v.at[2 * s]], rows_v.at[r], gsems[r])

    def wait_gather(r, s):
        pltpu.make_async_copy(
            tbl_sh.at[idx_v.at[2 * s]], rows_v.at[r], gsems[r]).wait()

    def start_scatter(r, s):
        pltpu.async_copy(
            rows_v.at[r], acc_sh.at[idx_v.at[2 * s + 1]], ssems[r], add=True)

    def wait_scatter(r, s):
        pltpu.make_async_copy(
            rows_v.at[r], acc_sh.at[idx_v.at[2 * s + 1]], ssems[r]).wait()

    for s in range(6):
        start_idx(s, s)
    for b in range(2):
        wait_idx(b, b)
        start_gather(b, b)

    def step(jj, statics):
        r, s = statics
        rn, sn, sf = (r + 2) % 4, (s + 2) % 8, (s + 6) % 8
        wait_gather(r, s)
        start_scatter(r, s)
        _maybe_when(jj >= 2, lambda: wait_scatter(rn, sf))
        _maybe_when(jj + 6 < NFC, lambda: start_idx(jj + 6, sf))

        def _pref():
            wait_idx(jj + 2, sn)
            start_gather(rn, sn)
        _maybe_when(jj + 2 < NFC, _pref)

    def oct_(t, _):
        j = t * 8
        for b in range(8):
            step(j + b, (b % 4, b))
        return 0
    lax.fori_loop(0, 72 // 8, oct_, 0)
    for jj in range(72, NFC):
        step(jj, (jj % 4, jj % 8))

    wait_scatter((NFC - 2) % 4, (NFC - 2) % 8)
    wait_scatter((NFC - 1) % 4, (NFC - 1) % 8)

    pltpu.sync_copy(ei_hbm.at[pl.ds(ebase + NFC * CH, REM)],
                    idx_v.at[0, pl.ds(0, REM)])
    pltpu.sync_copy(ei_hbm.at[pl.ds(E + ebase + NFC * CH, REM)],
                    idx_v.at[1, pl.ds(0, REM)])
    pltpu.sync_copy(tbl_sh.at[idx_v.at[0, pl.ds(0, REM)]], rem_v)
    pltpu.sync_copy(rem_v, acc_sh.at[idx_v.at[1, pl.ds(0, REM)]], add=True)

    plsc.subcore_barrier()
    base = sid * (NROWS // NS)
    pltpu.sync_copy(
        acc_sh.at[pl.ds(base, NROWS // NS)],
        out_hbm.at[pl.ds(cid * NROWS + base, NROWS // NS)],
    )


def _tc_call(body, out_shapes, *args):
    return pl.pallas_call(
        body,
        out_shape=out_shapes,
    )(*args)


def _tc1_body(x_ref, w1_ref, degp_ref, dinv_ref, hws_ref):
    deg = (degp_ref[0:N, 0:1]
           + degp_ref[NROWS:NROWS + N, 0:1] + 1.0)
    dinv = lax.rsqrt(deg)
    dinv_ref[...] = dinv
    hw = jnp.dot(x_ref[...], w1_ref[...], preferred_element_type=jnp.float32)
    hws_ref[...] = hw * dinv


def _tc_mid_body(parts_ref, hws_ref, dinv_ref, b_ref, w_ref, out_ref):
    dinv = dinv_ref[...]
    agg = parts_ref[0:N, :] + parts_ref[NROWS:NROWS + N, :] + hws_ref[...]
    h = jnp.maximum(agg * dinv + b_ref[...], 0.0)
    out_ref[...] = jnp.dot(h, w_ref[...],
                           preferred_element_type=jnp.float32) * dinv


def _sigmoid(x):
    return 1.0 / (1.0 + jnp.exp(-x))


def _tc_fin_body(parts_ref, hws_ref, dinv_ref, b3_ref,
                 wc1_ref, bc1_ref, wc2_ref, bc2_ref,
                 wr1_ref, br1_ref, wr2_ref, br2_ref,
                 h_ref, causal_ref, risk_ref):
    dinv = dinv_ref[...]
    agg = parts_ref[0:N, :] + parts_ref[NROWS:NROWS + N, :] + hws_ref[...]
    h = agg * dinv + b3_ref[...]
    h_ref[...] = h
    c1 = jnp.maximum(jnp.dot(h, wc1_ref[...],
                             preferred_element_type=jnp.float32)
                     + bc1_ref[...], 0.0)
    causal_ref[...] = _sigmoid(
        jnp.sum(c1 * wc2_ref[...], axis=1, keepdims=True) + bc2_ref[...])
    r1 = jnp.maximum(jnp.dot(h, wr1_ref[...],
                             preferred_element_type=jnp.float32)
                     + br1_ref[...], 0.0)
    risk_ref[...] = _sigmoid(
        jnp.sum(r1 * wr2_ref[...], axis=1, keepdims=True) + br2_ref[...])


def kernel(x, edge_index, W1, b1, W2, b2, W3, b3,
           Wc1, bc1, Wc2, bc2, Wr1, br1, Wr2, br2):
    ei = edge_index.reshape(2 * E)

    b1r = b1.reshape(1, H)
    b2r = b2.reshape(1, H)
    b3r = b3.reshape(1, H)
    bc1r = bc1.reshape(1, H // 2)
    bc2r = bc2.reshape(1, 1)
    br1r = br1.reshape(1, H // 2)
    br2r = br2.reshape(1, 1)
    wc2r = Wc2.reshape(1, H // 2)
    wr2r = Wr2.reshape(1, H // 2)

    degp = _deg_kernel()(ei)

    dinv, hws1 = _tc_call(
        _tc1_body,
        (jax.ShapeDtypeStruct((N, 1), jnp.float32),
         jax.ShapeDtypeStruct((N, H), jnp.float32)),
        x, W1, degp)

    parts1 = _agg_kernel()(hws1, ei)
    hws2 = _tc_call(
        _tc_mid_body,
        jax.ShapeDtypeStruct((N, H), jnp.float32),
        parts1, hws1, dinv, b1r, W2)

    parts2 = _agg_kernel()(hws2, ei)
    hws3 = _tc_call(
        _tc_mid_body,
        jax.ShapeDtypeStruct((N, H), jnp.float32),
        parts2, hws2, dinv, b2r, W3)

    parts3 = _agg_kernel()(hws3, ei)
    h, causal, risk = _tc_call(
        _tc_fin_body,
        (jax.ShapeDtypeStruct((N, H), jnp.float32),
         jax.ShapeDtypeStruct((N, 1), jnp.float32),
         jax.ShapeDtypeStruct((N, 1), jnp.float32)),
        parts3, hws3, dinv, b3r,
        Wc1, bc1r, wc2r, bc2r, Wr1, br1r, wr2r, br2r)

    return (h, causal, risk)
